# speedup vs baseline: 22.0930x; 22.0930x over previous
import jax
import jax.numpy as jnp
from jax.experimental import pallas as pl

N = 256
F = 255
E = 65536
HIDDEN = 2048
OUT_DIM = 32640


_EC = 2048
_NEC = E // _EC


def _c_kernel(src_ref, dst_ref, c_ref):
    i = pl.program_id(0)
    s = src_ref[0]
    d = dst_ref[0]
    rows = jax.lax.broadcasted_iota(jnp.int32, (N, _EC), 0)
    ohs = (rows == s).astype(jnp.float32)
    ohd = (rows == d).astype(jnp.float32)
    blk = jax.lax.dot_general(
        ohd, ohs, (((1,), (1,)), ((), ())),
        preferred_element_type=jnp.float32)

    @pl.when(i == 0)
    def _():
        rr = jax.lax.broadcasted_iota(jnp.int32, (N, N), 0)
        cc = jax.lax.broadcasted_iota(jnp.int32, (N, N), 1)
        c_ref[...] = blk + (rr == cc).astype(jnp.float32)

    @pl.when(i > 0)
    def _():
        c_ref[...] += blk


def _build_counts(src3, dst3):
    return pl.pallas_call(
        _c_kernel,
        grid=(_NEC,),
        in_specs=[
            pl.BlockSpec((1, 1, _EC), lambda i: (i, 0, 0)),
            pl.BlockSpec((1, 1, _EC), lambda i: (i, 0, 0)),
        ],
        out_specs=pl.BlockSpec((N, N), lambda i: (0, 0)),
        out_shape=jax.ShapeDtypeStruct((N, N), jnp.float32),
    )(src3, dst3)


def _gat_layer(h_in, C, mask, W, a_s_row, a_d_col, b_row):
    h = jnp.dot(h_in, W, preferred_element_type=jnp.float32)
    al_s = jax.lax.dot_general(
        a_s_row, h, (((1,), (1,)), ((), ())),
        preferred_element_type=jnp.float32)
    al_d = jnp.dot(h, a_d_col, preferred_element_type=jnp.float32)
    e = al_d + al_s
    e = jnp.where(e >= 0, e, 0.2 * e)
    em = jnp.where(mask, e, -1e30)
    m = jnp.max(em, axis=1, keepdims=True)
    p = jnp.exp(em - m) * C
    denom = jnp.sum(p, axis=1, keepdims=True)
    P = p / (denom + 1e-16)
    return jnp.dot(P, h, preferred_element_type=jnp.float32) + b_row


def _gat3_kernel(x_ref, c_ref,
                 w1_ref, as1_ref, ad1_ref, b1_ref,
                 w2_ref, as2_ref, ad2_ref, b2_ref,
                 w3_ref, as3_ref, ad3_ref, b3_ref,
                 out_ref):
    C = c_ref[...]
    mask = C > 0
    h = x_ref[...]
    h = _gat_layer(h, C, mask, w1_ref[...], as1_ref[...], ad1_ref[...],
                   b1_ref[...])
    h = _gat_layer(h, C, mask, w2_ref[...], as2_ref[...], ad2_ref[...],
                   b2_ref[...])
    h = _gat_layer(h, C, mask, w3_ref[...], as3_ref[...], ad3_ref[...],
                   b3_ref[...])
    out_ref[...] = jnp.maximum(h[:, :F], 0.0)


def _run_gat3(xp, C, layer_params):
    flat = []
    for (Wp, a_s, a_d, b) in layer_params:
        flat += [Wp, a_s, a_d, b]
    return pl.pallas_call(
        _gat3_kernel,
        out_shape=jax.ShapeDtypeStruct((N, F), jnp.float32),
    )(xp, C, *flat)


_K1 = 3840
_N1 = 512
_NK1 = (N * F) // _K1
_NN1 = HIDDEN // _N1


def _mv1_kernel(y_ref, w_ref, b_ref, o_ref):
    k = pl.program_id(1)
    part = jnp.dot(y_ref[...], w_ref[...], preferred_element_type=jnp.float32)

    @pl.when(k == 0)
    def _():
        o_ref[...] = part

    @pl.when(k > 0)
    def _():
        o_ref[...] += part

    @pl.when(k == _NK1 - 1)
    def _():
        o_ref[...] = jnp.maximum(o_ref[...] + b_ref[...], 0.0)


def _run_mv1(y0, W4, b4):
    return pl.pallas_call(
        _mv1_kernel,
        grid=(_NN1, _NK1),
        in_specs=[
            pl.BlockSpec((1, _K1), lambda n, k: (0, k)),
            pl.BlockSpec((_K1, _N1), lambda n, k: (k, n)),
            pl.BlockSpec((1, _N1), lambda n, k: (0, n)),
        ],
        out_specs=pl.BlockSpec((1, _N1), lambda n, k: (0, n)),
        out_shape=jax.ShapeDtypeStruct((1, HIDDEN), jnp.float32),
    )(y0, W4, b4)


_N2 = 1920
_NN2 = OUT_DIM // _N2


def _mv2_kernel(y_ref, w_ref, b_ref, o_ref):
    o_ref[...] = (
        jnp.dot(y_ref[...], w_ref[...], preferred_element_type=jnp.float32)
        + b_ref[...])


def _run_mv2(y1, W5, b5):
    return pl.pallas_call(
        _mv2_kernel,
        grid=(_NN2,),
        in_specs=[
            pl.BlockSpec((1, HIDDEN), lambda j: (0, 0)),
            pl.BlockSpec((HIDDEN, _N2), lambda j: (0, j)),
            pl.BlockSpec((1, _N2), lambda j: (0, j)),
        ],
        out_specs=pl.BlockSpec((1, _N2), lambda j: (0, j)),
        out_shape=jax.ShapeDtypeStruct((1, OUT_DIM), jnp.float32),
    )(y1, W5, b5)


def _pad_w(W):
    return jnp.pad(W, ((0, 1), (0, 1)))


def kernel(x, edge_index, W1, a_src1, a_dst1, b1, W2, a_src2, a_dst2, b2,
           W3, a_src3, a_dst3, b3, W4, b4, W5, b5):
    ei = edge_index.astype(jnp.int32)
    src3 = ei[0].reshape(_NEC, 1, _EC)
    dst3 = ei[1].reshape(_NEC, 1, _EC)
    C = _build_counts(src3, dst3)

    xp = jnp.pad(x, ((0, 0), (0, 1)))
    layer_params = []
    for (W, a_s, a_d, b) in ((W1, a_src1, a_dst1, b1),
                             (W2, a_src2, a_dst2, b2),
                             (W3, a_src3, a_dst3, b3)):
        layer_params.append((
            _pad_w(W),
            jnp.pad(a_s, (0, 1)).reshape(1, N),
            jnp.pad(a_d, (0, 1)).reshape(N, 1),
            jnp.pad(b, (0, 1)).reshape(1, N),
        ))

    g3r = _run_gat3(xp, C, layer_params)
    y0 = g3r.reshape(1, N * F)
    y1 = _run_mv1(y0, W4, b4.reshape(1, HIDDEN))
    y2 = _run_mv2(y1, W5, b5.reshape(1, OUT_DIM))
    return y2.reshape(OUT_DIM)
